# baseline (speedup 1.0000x reference)
import functools

import jax
import jax.numpy as jnp
from jax import lax
from jax.experimental import pallas as pl
from jax.experimental.pallas import tpu as pltpu
from jax.experimental.pallas import tpu_sc as plsc

B = 16384
D = 32
RW = 128
NC = 2
NS = 16
NW = NC * NS
BPW = B // NW
NCHUNK = 4
CHUNK = BPW // NCHUNK
GPC = CHUNK // 16
WPAD = 80


def _sc_body(users_hbm, movies_hbm, ut_hbm, mt_hbm, w_hbm, out_hbm,
             uidx, midx, uq, mq, ubuf0, ubuf1, mbuf0, mbuf1, wv, outv, sem):
    c = lax.axis_index("c")
    s = lax.axis_index("s")
    wid = s * NC + c
    base = wid * BPW

    for j in range(NCHUNK):
        pltpu.sync_copy(users_hbm.at[pl.ds(base + j * CHUNK, CHUNK)],
                        uidx.at[j])
        pltpu.sync_copy(movies_hbm.at[pl.ds(base + j * CHUNK, CHUNK)],
                        midx.at[j])
    pltpu.sync_copy(w_hbm, wv)

    for j in range(NCHUNK):
        for k in range(CHUNK // 16):
            uq[j, pl.ds(k * 16, 16)] = uidx[j, pl.ds(k * 16, 16)] >> 2
            mq[j, pl.ds(k * 16, 16)] = midx[j, pl.ds(k * 16, 16)] >> 2

    ubufs = [ubuf0, ubuf1]
    mbufs = [mbuf0, mbuf1]

    def fire(j):
        return (
            pltpu.async_copy(ut_hbm.at[uq.at[j]], ubufs[j % 2],
                             sem.at[2 * j]),
            pltpu.async_copy(mt_hbm.at[mq.at[j]], mbufs[j % 2],
                             sem.at[2 * j + 1]),
        )

    wvecs = [wv[pl.ds(k * 16, 16)] for k in range(5)]
    wsc = [wvecs[d // 16][d % 16] for d in range(2 * D)]
    bsc = wvecs[4][0]

    copies = {0: fire(0)}
    for j in range(NCHUNK):
        if j + 1 < NCHUNK:
            copies[j + 1] = fire(j + 1)
        cu, cm = copies[j]
        cu.wait()
        cm.wait()
        ub = ubufs[j % 2]
        mb = mbufs[j % 2]
        uref = uidx.at[j]
        mref = midx.at[j]

        def group(k, carry):
            row = jnp.arange(16, dtype=jnp.int32) + k * 16
            ucol = (uref[pl.ds(k * 16, 16)] & 3) * D
            mcol = (mref[pl.ds(k * 16, 16)] & 3) * D
            acc = jnp.zeros((16,), jnp.float32) + bsc
            for d in range(D):
                acc = acc + plsc.load_gather(ub, [row, ucol + d]) * wsc[d]
                acc = acc + plsc.load_gather(mb, [row, mcol + d]) * wsc[D + d]
            outv[pl.ds(j * CHUNK + k * 16, 16)] = acc
            return carry

        lax.fori_loop(0, GPC, group, 0)

    pltpu.sync_copy(outv, out_hbm.at[pl.ds(base, BPW)])


_sc_call = functools.partial(
    pl.kernel,
    out_type=jax.ShapeDtypeStruct((B,), jnp.float32),
    mesh=plsc.VectorSubcoreMesh(core_axis_name="c", subcore_axis_name="s"),
    scratch_types=[
        pltpu.VMEM((NCHUNK, CHUNK), jnp.int32),
        pltpu.VMEM((NCHUNK, CHUNK), jnp.int32),
        pltpu.VMEM((NCHUNK, CHUNK), jnp.int32),
        pltpu.VMEM((NCHUNK, CHUNK), jnp.int32),
        pltpu.VMEM((CHUNK, RW), jnp.float32),
        pltpu.VMEM((CHUNK, RW), jnp.float32),
        pltpu.VMEM((CHUNK, RW), jnp.float32),
        pltpu.VMEM((CHUNK, RW), jnp.float32),
        pltpu.VMEM((WPAD,), jnp.float32),
        pltpu.VMEM((BPW,), jnp.float32),
        pltpu.SemaphoreType.DMA((2 * NCHUNK,)),
    ],
    compiler_params=pltpu.CompilerParams(needs_layout_passes=False),
)(_sc_body)


@jax.jit
def kernel(users, movies, user_table, movie_table, W, b):
    wvec = jnp.zeros((WPAD,), jnp.float32)
    wvec = wvec.at[: 2 * D].set(W.reshape(-1))
    wvec = wvec.at[2 * D].set(b[0])
    ut = user_table.reshape(-1, RW)
    mt = movie_table.reshape(-1, RW)
    out = _sc_call(users.astype(jnp.int32), movies.astype(jnp.int32),
                   ut, mt, wvec)
    return out.reshape(B, 1)


# speedup vs baseline: 8.6341x; 8.6341x over previous
import functools

import jax
import jax.numpy as jnp
from jax import lax
from jax.experimental import pallas as pl
from jax.experimental.pallas import tpu as pltpu
from jax.experimental.pallas import tpu_sc as plsc

B = 16384
D = 32
NROWS = 1000000
BC = 32768
NB = (NROWS + BC - 1) // BC
NC = 2
NS = 16
NW = NC * NS
BPW = B // NW
NCHUNK = 4
CHUNK = BPW // NCHUNK
VPC = CHUNK // 16


def _proj_body(w_ref, ut_ref, mt_ref, pu_ref, pm_ref):
    wu = w_ref[0, :D].reshape(D, 1)
    wm = w_ref[0, D:].reshape(D, 1)
    pu_ref[...] = jnp.sum(ut_ref[...] * wu, axis=0)
    pm_ref[...] = jnp.sum(mt_ref[...] * wm, axis=0)


_proj_call = pl.pallas_call(
    _proj_body,
    grid=(NB,),
    in_specs=[
        pl.BlockSpec((1, 2 * D), lambda i: (0, 0)),
        pl.BlockSpec((D, BC), lambda i: (0, i)),
        pl.BlockSpec((D, BC), lambda i: (0, i)),
    ],
    out_specs=[
        pl.BlockSpec((BC,), lambda i: (i,)),
        pl.BlockSpec((BC,), lambda i: (i,)),
    ],
    out_shape=[
        jax.ShapeDtypeStruct((NROWS,), jnp.float32),
        jax.ShapeDtypeStruct((NROWS,), jnp.float32),
    ],
)


def _gather_body(users_hbm, movies_hbm, pu_hbm, pm_hbm, b_hbm, out_hbm,
                 uidx, midx, ubuf, mbuf, bv, outv, sems):
    c = lax.axis_index("c")
    s = lax.axis_index("s")
    wid = s * NC + c
    base = wid * BPW

    for j in range(NCHUNK):
        pltpu.sync_copy(users_hbm.at[pl.ds(base + j * CHUNK, CHUNK)],
                        uidx.at[j])
        pltpu.sync_copy(movies_hbm.at[pl.ds(base + j * CHUNK, CHUNK)],
                        midx.at[j])
    pltpu.sync_copy(b_hbm, bv)
    bsc = bv[pl.ds(0, 16)][0]

    def fire(j, ph):
        return (
            pltpu.async_copy(pu_hbm.at[uidx.at[j]], ubuf.at[ph],
                             sems.at[0, ph]),
            pltpu.async_copy(pm_hbm.at[midx.at[j]], mbuf.at[ph],
                             sems.at[1, ph]),
        )

    cps = fire(0, 0)
    for j in range(NCHUNK):
        nxt = fire(j + 1, (j + 1) % 2) if j + 1 < NCHUNK else None
        cps[0].wait()
        cps[1].wait()
        ph = j % 2
        for k in range(VPC):
            outv[pl.ds(j * CHUNK + k * 16, 16)] = (
                ubuf[ph, pl.ds(k * 16, 16)]
                + mbuf[ph, pl.ds(k * 16, 16)] + bsc)
        cps = nxt

    pltpu.sync_copy(outv, out_hbm.at[pl.ds(base, BPW)])


_gather_call = functools.partial(
    pl.kernel,
    out_type=jax.ShapeDtypeStruct((B,), jnp.float32),
    mesh=plsc.VectorSubcoreMesh(core_axis_name="c", subcore_axis_name="s"),
    scratch_types=[
        pltpu.VMEM((NCHUNK, CHUNK), jnp.int32),
        pltpu.VMEM((NCHUNK, CHUNK), jnp.int32),
        pltpu.VMEM((2, CHUNK), jnp.float32),
        pltpu.VMEM((2, CHUNK), jnp.float32),
        pltpu.VMEM((16,), jnp.float32),
        pltpu.VMEM((BPW,), jnp.float32),
        pltpu.SemaphoreType.DMA((2, 2)),
    ],
    compiler_params=pltpu.CompilerParams(
        needs_layout_passes=False, use_tc_tiling_on_sc=False),
)(_gather_body)


@jax.jit
def kernel(users, movies, user_table, movie_table, W, b):
    pu, pm = _proj_call(W, user_table.T, movie_table.T)
    bpad = jnp.zeros((16,), jnp.float32).at[0].set(b[0])
    out = _gather_call(users.astype(jnp.int32), movies.astype(jnp.int32),
                       pu, pm, bpad)
    return out.reshape(B, 1)
